# baseline (speedup 1.0000x reference)
import functools

import jax
import jax.numpy as jnp
from jax import lax
from jax.experimental import pallas as pl
from jax.experimental.pallas import tpu as pltpu
from jax.experimental.pallas import tpu_sc as plsc

NC = 2
NS = 16
L = 16
NW = NC * NS

BATCH = 16384
EMBED_K = 16
BPW = BATCH // NW
NROW = 4
IDXW = BPW // NROW


def _sc_body(x_hbm, w_hbm, h_hbm, out_hbm, xv, uidx, iidx, uv, vv, outv, sem):
    wid = lax.axis_index("s") * NC + lax.axis_index("c")
    base = wid * BPW

    pltpu.sync_copy(x_hbm.at[pl.ds(base * 2, BPW * 2)], xv)

    lane = lax.iota(jnp.int32, L)
    zeros = jnp.zeros((L,), jnp.int32)

    def deint_body(g, carry):
        rows = g * L + lane
        u = plsc.load_gather(xv, [rows * 2])
        it = plsc.load_gather(xv, [rows * 2 + 1])
        r = g >> 3
        c = (g & 7) * L
        uidx[r, pl.ds(c, L)] = u
        iidx[r, pl.ds(c, L)] = it
        return carry

    lax.fori_loop(0, BPW // L, deint_body, jnp.int32(0))

    copies = []
    for j in range(NROW):
        copies.append(pltpu.async_copy(
            w_hbm.at[uidx.at[j]], uv.at[pl.ds(j * IDXW, IDXW)], sem))
        copies.append(pltpu.async_copy(
            h_hbm.at[iidx.at[j]], vv.at[pl.ds(j * IDXW, IDXW)], sem))
    for c in copies:
        c.wait()

    def group_body(g, carry):
        rows = g * L + lane
        acc = jnp.zeros((L,), jnp.float32)
        for k in range(EMBED_K):
            cols = zeros + k
            uk = plsc.load_gather(uv, [rows, cols])
            vk = plsc.load_gather(vv, [rows, cols])
            acc = acc + uk * vk
        outv[pl.ds(g * L, L)] = acc
        return carry

    lax.fori_loop(0, BPW // L, group_body, jnp.int32(0))

    pltpu.sync_copy(outv, out_hbm.at[pl.ds(base, BPW)])


@jax.jit
def _mf_dot(x, W, H):
    mesh = plsc.VectorSubcoreMesh(core_axis_name="c", subcore_axis_name="s")
    return pl.kernel(
        _sc_body,
        out_type=jax.ShapeDtypeStruct((BATCH,), jnp.float32),
        mesh=mesh,
        compiler_params=pltpu.CompilerParams(
            needs_layout_passes=False, use_tc_tiling_on_sc=False),
        scratch_types=[
            pltpu.VMEM((BPW * 2,), jnp.int32),
            pltpu.VMEM((NROW, IDXW), jnp.int32),
            pltpu.VMEM((NROW, IDXW), jnp.int32),
            pltpu.VMEM((BPW, EMBED_K), jnp.float32),
            pltpu.VMEM((BPW, EMBED_K), jnp.float32),
            pltpu.VMEM((BPW,), jnp.float32),
            pltpu.SemaphoreType.DMA,
        ],
    )(x, W, H)


def kernel(x, W, H):
    return _mf_dot(x.astype(jnp.int32).reshape(-1), W, H)


# speedup vs baseline: 6.3574x; 6.3574x over previous
import functools

import jax
import jax.numpy as jnp
from jax import lax
from jax.experimental import pallas as pl
from jax.experimental.pallas import tpu as pltpu
from jax.experimental.pallas import tpu_sc as plsc

NC = 2
NS = 16
L = 16
NW = NC * NS

BATCH = 16384
EMBED_K = 16
BPW = BATCH // NW
C = 8
NCHUNK = BPW // C


def _sc_body(ui_hbm, ii_hbm, wt_hbm, ht_hbm, out_hbm,
             uiv, iiv, ubuf, vbuf, outv, semu, semv):
    wid = lax.axis_index("s") * NC + lax.axis_index("c")
    base = wid * BPW

    pltpu.sync_copy(ui_hbm.at[pl.ds(base, BPW)], uiv)
    pltpu.sync_copy(ii_hbm.at[pl.ds(base, BPW)], iiv)

    lanes = lax.iota(jnp.int32, L)

    def pair_vecs(p):
        return uiv[pl.ds(p * L, L)], iiv[pl.ds(p * L, L)]

    def fire(uvec, ivec, h, slot):
        for k in range(C):
            r = uvec[h + k]
            rb = pl.multiple_of(r & ~127, 128)
            pltpu.async_copy(
                wt_hbm.at[:, pl.ds(rb, 128)], ubuf.at[slot, k], semu.at[slot])
            q = ivec[h + k]
            qb = pl.multiple_of(q & ~127, 128)
            pltpu.async_copy(
                ht_hbm.at[:, pl.ds(qb, 128)], vbuf.at[slot, k], semv.at[slot])

    def drain(slot):
        for k in range(C):
            pltpu.make_async_copy(
                wt_hbm.at[:, pl.ds(0, 128)], ubuf.at[slot, k],
                semu.at[slot]).wait()
            pltpu.make_async_copy(
                ht_hbm.at[:, pl.ds(0, 128)], vbuf.at[slot, k],
                semv.at[slot]).wait()

    def compute(uvec, ivec, h, slot, dots):
        for k in range(C):
            rm = uvec[h + k] & 127
            qm = ivec[h + k] & 127
            u = plsc.load_gather(
                ubuf.at[slot, k], [lanes, jnp.zeros((L,), jnp.int32) + rm])
            v = plsc.load_gather(
                vbuf.at[slot, k], [lanes, jnp.zeros((L,), jnp.int32) + qm])
            s = jnp.sum(u * v)
            dots = jnp.where(lanes == h + k, s, dots)
        return dots

    uvec0, ivec0 = pair_vecs(0)
    fire(uvec0, ivec0, 0, 0)

    def pair_body(p, carry):
        uvec, ivec = pair_vecs(p)
        fire(uvec, ivec, C, 1)
        drain(0)
        dots = jnp.zeros((L,), jnp.float32)
        dots = compute(uvec, ivec, 0, 0, dots)

        @pl.when(p + 1 < BPW // L)
        def _():
            nuvec, nivec = pair_vecs(p + 1)
            fire(nuvec, nivec, 0, 0)

        drain(1)
        dots = compute(uvec, ivec, C, 1, dots)
        outv[pl.ds(p * L, L)] = dots
        return carry

    lax.fori_loop(0, BPW // L, pair_body, jnp.int32(0))

    pltpu.sync_copy(outv, out_hbm.at[pl.ds(base, BPW)])


@jax.jit
def _mf_dot(x, W, H):
    ui = x[:, 0]
    ii = x[:, 1]
    wt = W.T
    ht = H.T
    mesh = plsc.VectorSubcoreMesh(core_axis_name="c", subcore_axis_name="s")
    return pl.kernel(
        _sc_body,
        out_type=jax.ShapeDtypeStruct((BATCH,), jnp.float32),
        mesh=mesh,
        compiler_params=pltpu.CompilerParams(
            needs_layout_passes=False, use_tc_tiling_on_sc=True),
        scratch_types=[
            pltpu.VMEM((BPW,), jnp.int32),
            pltpu.VMEM((BPW,), jnp.int32),
            pltpu.VMEM((2, C, EMBED_K, 128), jnp.float32),
            pltpu.VMEM((2, C, EMBED_K, 128), jnp.float32),
            pltpu.VMEM((BPW,), jnp.float32),
            pltpu.SemaphoreType.DMA((2,)),
            pltpu.SemaphoreType.DMA((2,)),
        ],
    )(ui, ii, wt, ht)


def kernel(x, W, H):
    return _mf_dot(x.astype(jnp.int32), W, H)


# speedup vs baseline: 6.9664x; 1.0958x over previous
import functools

import jax
import jax.numpy as jnp
from jax import lax
from jax.experimental import pallas as pl
from jax.experimental.pallas import tpu as pltpu
from jax.experimental.pallas import tpu_sc as plsc

NC = 2
NS = 16
L = 16
NW = NC * NS

BATCH = 16384
EMBED_K = 16
BPW = BATCH // NW
C = 8
SLOTS = 3
NCHUNK = BPW // C


def _sc_body(ui_hbm, ii_hbm, wt_hbm, ht_hbm, out_hbm,
             uiv, iiv, ubuf, vbuf, outv, semu, semv):
    wid = lax.axis_index("s") * NC + lax.axis_index("c")
    base = wid * BPW

    pltpu.sync_copy(ui_hbm.at[pl.ds(base, BPW)], uiv.at[pl.ds(0, BPW)])
    pltpu.sync_copy(ii_hbm.at[pl.ds(base, BPW)], iiv.at[pl.ds(0, BPW)])

    lanes = lax.iota(jnp.int32, L)

    def chunk_vecs(c):
        return uiv[pl.ds(c * C, L)], iiv[pl.ds(c * C, L)]

    def fire(c, slot):
        uvec, ivec = chunk_vecs(c)
        for k in range(C):
            r = uvec[k]
            rb = pl.multiple_of(r & ~127, 128)
            pltpu.async_copy(
                wt_hbm.at[:, pl.ds(rb, 128)], ubuf.at[slot, k], semu.at[slot])
            q = ivec[k]
            qb = pl.multiple_of(q & ~127, 128)
            pltpu.async_copy(
                ht_hbm.at[:, pl.ds(qb, 128)], vbuf.at[slot, k], semv.at[slot])

    def drain(slot):
        for k in range(C):
            pltpu.make_async_copy(
                wt_hbm.at[:, pl.ds(0, 128)], ubuf.at[slot, k],
                semu.at[slot]).wait()
            pltpu.make_async_copy(
                ht_hbm.at[:, pl.ds(0, 128)], vbuf.at[slot, k],
                semv.at[slot]).wait()

    def compute(c, slot, off):
        uvec, ivec = chunk_vecs(c)
        dots = jnp.zeros((L,), jnp.float32)
        for k in range(C):
            rm = uvec[k] & 127
            qm = ivec[k] & 127
            u = plsc.load_gather(
                ubuf.at[slot, k], [lanes, jnp.zeros((L,), jnp.int32) + rm])
            v = plsc.load_gather(
                vbuf.at[slot, k], [lanes, jnp.zeros((L,), jnp.int32) + qm])
            s = jnp.sum(u * v)
            dots = jnp.where(lanes == off + k, s, dots)
        return dots

    fire(0, 0)
    fire(1, 1)

    def chunk_body(c, carry):
        slot = lax.rem(c, SLOTS)

        @pl.when(c + 2 < NCHUNK)
        def _():
            fire(c + 2, lax.rem(c + 2, SLOTS))

        drain(slot)
        blk = (c >> 1) * L
        h = (c & 1) * C
        dots = compute(c, slot, h)

        ob = outv[pl.ds(blk, L)]
        ob = jnp.where((lanes >= h) & (lanes < h + C), dots, ob)
        outv[pl.ds(blk, L)] = ob
        return carry

    lax.fori_loop(0, NCHUNK, chunk_body, jnp.int32(0))

    pltpu.sync_copy(outv, out_hbm.at[pl.ds(base, BPW)])


@jax.jit
def _mf_dot(x, W, H):
    ui = x[:, 0]
    ii = x[:, 1]
    wt = W.T
    ht = H.T
    mesh = plsc.VectorSubcoreMesh(core_axis_name="c", subcore_axis_name="s")
    return pl.kernel(
        _sc_body,
        out_type=jax.ShapeDtypeStruct((BATCH,), jnp.float32),
        mesh=mesh,
        compiler_params=pltpu.CompilerParams(
            needs_layout_passes=False, use_tc_tiling_on_sc=True),
        scratch_types=[
            pltpu.VMEM((BPW + L,), jnp.int32),
            pltpu.VMEM((BPW + L,), jnp.int32),
            pltpu.VMEM((SLOTS, C, EMBED_K, 128), jnp.float32),
            pltpu.VMEM((SLOTS, C, EMBED_K, 128), jnp.float32),
            pltpu.VMEM((BPW,), jnp.float32),
            pltpu.SemaphoreType.DMA((SLOTS,)),
            pltpu.SemaphoreType.DMA((SLOTS,)),
        ],
    )(ui, ii, wt, ht)


def kernel(x, W, H):
    return _mf_dot(x.astype(jnp.int32), W, H)


# speedup vs baseline: 7.1029x; 1.0196x over previous
import functools

import jax
import jax.numpy as jnp
from jax import lax
from jax.experimental import pallas as pl
from jax.experimental.pallas import tpu as pltpu
from jax.experimental.pallas import tpu_sc as plsc

NC = 2
NS = 16
L = 16
NW = NC * NS

BATCH = 16384
EMBED_K = 16
BPW = BATCH // NW
C = 4
SLOTS = 6
NCHUNK = BPW // C


def _sc_body(ui_hbm, ii_hbm, wt_hbm, ht_hbm, out_hbm,
             uiv, iiv, ubuf, vbuf, outv, semu, semv):
    wid = lax.axis_index("s") * NC + lax.axis_index("c")
    base = wid * BPW

    pltpu.sync_copy(ui_hbm.at[pl.ds(base, BPW)], uiv.at[pl.ds(0, BPW)])
    pltpu.sync_copy(ii_hbm.at[pl.ds(base, BPW)], iiv.at[pl.ds(0, BPW)])

    lanes = lax.iota(jnp.int32, L)

    def chunk_vecs(c):
        return uiv[pl.ds(c * C, L)], iiv[pl.ds(c * C, L)]

    def fire(c, slot):
        uvec, ivec = chunk_vecs(c)
        for k in range(C):
            r = uvec[k]
            rb = pl.multiple_of(r & ~127, 128)
            pltpu.async_copy(
                wt_hbm.at[:, pl.ds(rb, 128)], ubuf.at[slot, k], semu.at[slot])
            q = ivec[k]
            qb = pl.multiple_of(q & ~127, 128)
            pltpu.async_copy(
                ht_hbm.at[:, pl.ds(qb, 128)], vbuf.at[slot, k], semv.at[slot])

    def drain(slot):
        for k in range(C):
            pltpu.make_async_copy(
                wt_hbm.at[:, pl.ds(0, 128)], ubuf.at[slot, k],
                semu.at[slot]).wait()
            pltpu.make_async_copy(
                ht_hbm.at[:, pl.ds(0, 128)], vbuf.at[slot, k],
                semv.at[slot]).wait()

    def compute(c, slot, off):
        uvec, ivec = chunk_vecs(c)
        dots = jnp.zeros((L,), jnp.float32)
        for k in range(C):
            rm = uvec[k] & 127
            qm = ivec[k] & 127
            u = plsc.load_gather(
                ubuf.at[slot, k], [lanes, jnp.zeros((L,), jnp.int32) + rm])
            v = plsc.load_gather(
                vbuf.at[slot, k], [lanes, jnp.zeros((L,), jnp.int32) + qm])
            s = jnp.sum(u * v)
            dots = jnp.where(lanes == off + k, s, dots)
        return dots

    AHEAD = SLOTS - 1
    CPB = L // C
    for j in range(AHEAD):
        fire(j, j)

    def chunk_body(c, carry):
        slot = lax.rem(c, SLOTS)

        @pl.when(c + AHEAD < NCHUNK)
        def _():
            fire(c + AHEAD, lax.rem(c + AHEAD, SLOTS))

        drain(slot)
        blk = (c // CPB) * L
        h = lax.rem(c, CPB) * C
        dots = compute(c, slot, h)

        ob = outv[pl.ds(blk, L)]
        ob = jnp.where((lanes >= h) & (lanes < h + C), dots, ob)
        outv[pl.ds(blk, L)] = ob
        return carry

    lax.fori_loop(0, NCHUNK, chunk_body, jnp.int32(0))

    pltpu.sync_copy(outv, out_hbm.at[pl.ds(base, BPW)])


@jax.jit
def _mf_dot(x, W, H):
    ui = x[:, 0]
    ii = x[:, 1]
    wt = W.T
    ht = H.T
    mesh = plsc.VectorSubcoreMesh(core_axis_name="c", subcore_axis_name="s")
    return pl.kernel(
        _sc_body,
        out_type=jax.ShapeDtypeStruct((BATCH,), jnp.float32),
        mesh=mesh,
        compiler_params=pltpu.CompilerParams(
            needs_layout_passes=False, use_tc_tiling_on_sc=True),
        scratch_types=[
            pltpu.VMEM((BPW + L,), jnp.int32),
            pltpu.VMEM((BPW + L,), jnp.int32),
            pltpu.VMEM((SLOTS, C, EMBED_K, 128), jnp.float32),
            pltpu.VMEM((SLOTS, C, EMBED_K, 128), jnp.float32),
            pltpu.VMEM((BPW,), jnp.float32),
            pltpu.SemaphoreType.DMA((SLOTS,)),
            pltpu.SemaphoreType.DMA((SLOTS,)),
        ],
    )(ui, ii, wt, ht)


def kernel(x, W, H):
    return _mf_dot(x.astype(jnp.int32), W, H)
